# baseline (speedup 1.0000x reference)
import functools

import jax
import jax.numpy as jnp
from jax.experimental import pallas as pl
from jax.experimental.pallas import tpu as pltpu

K = 16
NBX = 1024
NBF = 2048
CAP = 4
CLS = 16
SROWS = 8
BIG = 3.0e37


def _batcher_pairs(n):
    pairs = []
    p = 1
    while p < n:
        k = p
        while k >= 1:
            for j in range(k % p, n - k, 2 * k):
                for i in range(0, min(k, n - j - k)):
                    if (i + j) // (2 * p) == (i + j + k) // (2 * p):
                        pairs.append((i + j, i + j + k))
            k //= 2
        p *= 2
    return pairs


_PAIRS16 = tuple(_batcher_pairs(16))
_PAIRS4 = tuple(_batcher_pairs(4))


def _ce(v, x, i, j):
    p = v[i] <= v[j]
    vi = jnp.where(p, v[i], v[j])
    vj = jnp.where(p, v[j], v[i])
    xi = jnp.where(p, x[i], x[j])
    xj = jnp.where(p, x[j], x[i])
    v[i], v[j], x[i], x[j] = vi, vj, xi, xj


def _merge_keep(av, ai, bv, bi):
    n = len(av)
    cv, ci = [], []
    for i in range(n):
        p = av[i] <= bv[n - 1 - i]
        cv.append(jnp.where(p, av[i], bv[n - 1 - i]))
        ci.append(jnp.where(p, ai[i], bi[n - 1 - i]))
    hh = n // 2
    while hh >= 1:
        for i in range(n):
            if i % (2 * hh) < hh:
                _ce(cv, ci, i, i + hh)
        hh //= 2
    return cv, ci


def _merge_full(av, ai, bv, bi):
    n = len(av)
    cv = av + bv[::-1]
    ci = ai + bi[::-1]
    hh = n
    while hh >= 1:
        for i in range(2 * n):
            if i % (2 * hh) < hh:
                _ce(cv, ci, i, i + hh)
        hh //= 2
    return cv, ci


def _halve(vals, idxs, full=False):
    rows = vals[0].shape[0]
    h = rows // 2
    merge = _merge_full if full else _merge_keep
    mv, mi = merge([v[:h] for v in vals], [x[:h] for x in idxs],
                   [v[h:2 * h] for v in vals], [x[h:2 * h] for x in idxs])
    if rows % 2:
        if full:
            raise ValueError("odd rows unsupported for full merge")
        mv = [jnp.concatenate([v, vals[i][2 * h:]], axis=0)
              for i, v in enumerate(mv)]
        mi = [jnp.concatenate([x, idxs[i][2 * h:]], axis=0)
              for i, x in enumerate(mi)]
    return mv, mi


def _block_planes(nparts, n_real, m, refb, qt, r2, q2, base):
    cross = jax.lax.dot_general(
        refb[...], qt[...], (((0,), (0,)), ((), ())),
        preferred_element_type=jnp.float32)
    nb = cross.shape[0]
    g = nb // nparts
    q2b = q2[...]
    giota = jax.lax.broadcasted_iota(jnp.int32, (g, m), 0)
    vals, idxs = [], []
    for r in range(nparts):
        lo = r * g
        d2r = (q2b + r2[lo:lo + g, :]) - 2.0 * cross[lo:lo + g, :]
        ir = giota + (base + lo)
        if n_real is not None:
            d2r = jnp.where(ir >= n_real, BIG, d2r)
        vals.append(d2r)
        idxs.append(ir)
    return vals, idxs


_LCH = 2


def _fast_body(nblocks, n_real, m, refb, qt, r2, q2, *outs):
    pi = pl.program_id(0)
    ov, oi = outs[:CAP], outs[CAP:]
    base = pi * NBF
    mc = m // _LCH
    g = NBF // CAP
    refbv = refb[...]
    qtv = qt[...]
    r2v = r2[...]
    q2v = q2[...]
    giota = jax.lax.broadcasted_iota(jnp.int32, (g, mc), 0)

    def run(mask_n):
        for ch in range(_LCH):
            lo_m = ch * mc
            cross = jax.lax.dot_general(
                refbv, qtv[:, lo_m:lo_m + mc], (((0,), (0,)), ((), ())),
                preferred_element_type=jnp.float32)
            q2c = q2v[:, lo_m:lo_m + mc]
            vals, idxs = [], []
            for r in range(CAP):
                lo = r * g
                d2r = (q2c + r2v[lo:lo + g, :]) - 2.0 * cross[lo:lo + g, :]
                ir = giota + (base + lo)
                if mask_n is not None:
                    d2r = jnp.where(ir >= mask_n, BIG, d2r)
                vals.append(d2r)
                idxs.append(ir)
            for (i, j) in _PAIRS4:
                _ce(vals, idxs, i, j)
            while vals[0].shape[0] > CLS:
                vals, idxs = _halve(vals, idxs)
            for r in range(CAP):
                ov[r][:, lo_m:lo_m + mc] = vals[r]
                oi[r][:, lo_m:lo_m + mc] = idxs[r]

    @pl.when(pi < nblocks - 1)
    def _():
        run(None)

    @pl.when(pi == nblocks - 1)
    def _():
        run(n_real)


def _collapse_body(m, iv0, iv1, iv2, iv3, ii0, ii1, ii2, ii3,
                   dout, iout, fout):
    av = [iv0[...], iv1[...], iv2[...], iv3[...]]
    ai = [ii0[...], ii1[...], ii2[...], ii3[...]]
    av, ai = _halve(av, ai, full=True)
    av, ai = _halve(av, ai, full=True)
    while av[0].shape[0] > 1:
        av, ai = _halve(av, ai)
    tau = av[K - 1]
    dout[...] = jnp.concatenate(
        [jnp.sqrt(jnp.maximum(v, 0.0)) for v in av], axis=0)
    iout[...] = jnp.concatenate(ai, axis=0)
    fire = (iv3[...] <= tau).astype(jnp.int32)
    fout[...] = jnp.max(fire, axis=0, keepdims=True)


def _exact_body(nblocks, g, n_real, m, refb, qt, r2, q2, dout, iout, sv, si):
    pi = pl.program_id(0)
    vals, idxs = _block_planes(K, n_real, m, refb, qt, r2, q2, pi * NBX)
    for (i, j) in _PAIRS16:
        _ce(vals, idxs, i, j)
    while vals[0].shape[0] > SROWS:
        vals, idxs = _halve(vals, idxs)

    @pl.when(pi == 0)
    def _():
        for i in range(K):
            sv[i * SROWS:(i + 1) * SROWS, :] = vals[i]
            si[i * SROWS:(i + 1) * SROWS, :] = idxs[i]

    @pl.when(pi > 0)
    def _():
        av = [sv[i * SROWS:(i + 1) * SROWS, :] for i in range(K)]
        ai = [si[i * SROWS:(i + 1) * SROWS, :] for i in range(K)]
        mv, mi = _merge_keep(av, ai, vals, idxs)
        for i in range(K):
            sv[i * SROWS:(i + 1) * SROWS, :] = mv[i]
            si[i * SROWS:(i + 1) * SROWS, :] = mi[i]

    @pl.when(pi == nblocks - 1)
    def _():
        fv = [sv[i * SROWS:(i + 1) * SROWS, :] for i in range(K)]
        fi = [si[i * SROWS:(i + 1) * SROWS, :] for i in range(K)]
        for (i, j) in _PAIRS16:
            _ce(fv, fi, i, j)
        while fv[0].shape[0] > 1:
            fv, fi = _halve(fv, fi)
        dout[...] = jnp.concatenate(
            [jnp.sqrt(jnp.maximum(v, 0.0)) for v in fv], axis=0)
        iout[...] = jnp.concatenate(fi, axis=0)


def kernel(ref, query):
    b, c, n = ref.shape
    m = query.shape[2]
    nbf = -(-n // NBF)
    nbx = -(-n // NBX)
    srows = nbf * CLS

    rp = jnp.transpose(ref, (0, 2, 1))
    qp = jnp.transpose(query, (0, 2, 1))
    r2col = jnp.sum(rp * rp, axis=-1)[0][:, None]
    q2row = jnp.sum(qp * qp, axis=-1)
    refsq = ref[0]
    qt = query[0]

    vmem = pltpu.CompilerParams(vmem_limit_bytes=63 * 1024 * 1024)
    state = pl.pallas_call(
        functools.partial(_fast_body, nbf, n, m),
        grid=(nbf,),
        compiler_params=vmem,
        in_specs=[
            pl.BlockSpec((c, NBF), lambda i: (0, i)),
            pl.BlockSpec((c, m), lambda i: (0, 0)),
            pl.BlockSpec((NBF, 1), lambda i: (i, 0)),
            pl.BlockSpec((1, m), lambda i: (0, 0)),
        ],
        out_specs=[pl.BlockSpec((CLS, m), lambda i: (i, 0))] * (2 * CAP),
        out_shape=(
            [jax.ShapeDtypeStruct((srows, m), jnp.float32)] * CAP
            + [jax.ShapeDtypeStruct((srows, m), jnp.int32)] * CAP),
    )(refsq, qt, r2col, q2row)

    dist_f, idx_f, flags = pl.pallas_call(
        functools.partial(_collapse_body, m),
        compiler_params=vmem,
        out_shape=[
            jax.ShapeDtypeStruct((K, m), jnp.float32),
            jax.ShapeDtypeStruct((K, m), jnp.int32),
            jax.ShapeDtypeStruct((1, m), jnp.int32),
        ],
    )(*state)

    def _exact():
        return pl.pallas_call(
            functools.partial(_exact_body, nbx, NBX // K, n, m),
            grid=(nbx,),
            compiler_params=vmem,
            in_specs=[
                pl.BlockSpec((c, NBX), lambda i: (0, i)),
                pl.BlockSpec((c, m), lambda i: (0, 0)),
                pl.BlockSpec((NBX, 1), lambda i: (i, 0)),
                pl.BlockSpec((1, m), lambda i: (0, 0)),
            ],
            out_specs=[
                pl.BlockSpec((K, m), lambda i: (0, 0)),
                pl.BlockSpec((K, m), lambda i: (0, 0)),
            ],
            out_shape=[
                jax.ShapeDtypeStruct((K, m), jnp.float32),
                jax.ShapeDtypeStruct((K, m), jnp.int32),
            ],
            scratch_shapes=[
                pltpu.VMEM((K * SROWS, m), jnp.float32),
                pltpu.VMEM((K * SROWS, m), jnp.int32),
            ],
        )(refsq, qt, r2col, q2row)

    dist, idx = jax.lax.cond(jnp.any(flags > 0), _exact,
                             lambda: (dist_f, idx_f))
    return dist[None], idx[None]


# speedup vs baseline: 1.0796x; 1.0796x over previous
import functools

import jax
import jax.numpy as jnp
from jax.experimental import pallas as pl
from jax.experimental.pallas import tpu as pltpu

K = 16
NBX = 1024
NBF = 2048
CAP = 4
CLS = 8
SROWS = 8
BIG = 3.0e37


def _batcher_pairs(n):
    pairs = []
    p = 1
    while p < n:
        k = p
        while k >= 1:
            for j in range(k % p, n - k, 2 * k):
                for i in range(0, min(k, n - j - k)):
                    if (i + j) // (2 * p) == (i + j + k) // (2 * p):
                        pairs.append((i + j, i + j + k))
            k //= 2
        p *= 2
    return pairs


_PAIRS16 = tuple(_batcher_pairs(16))
_PAIRS4 = tuple(_batcher_pairs(4))


def _ce(v, x, i, j):
    p = v[i] <= v[j]
    vi = jnp.where(p, v[i], v[j])
    vj = jnp.where(p, v[j], v[i])
    xi = jnp.where(p, x[i], x[j])
    xj = jnp.where(p, x[j], x[i])
    v[i], v[j], x[i], x[j] = vi, vj, xi, xj


def _merge_keep(av, ai, bv, bi):
    n = len(av)
    cv, ci = [], []
    for i in range(n):
        p = av[i] <= bv[n - 1 - i]
        cv.append(jnp.where(p, av[i], bv[n - 1 - i]))
        ci.append(jnp.where(p, ai[i], bi[n - 1 - i]))
    hh = n // 2
    while hh >= 1:
        for i in range(n):
            if i % (2 * hh) < hh:
                _ce(cv, ci, i, i + hh)
        hh //= 2
    return cv, ci


def _merge_full(av, ai, bv, bi):
    n = len(av)
    cv = av + bv[::-1]
    ci = ai + bi[::-1]
    hh = n
    while hh >= 1:
        for i in range(2 * n):
            if i % (2 * hh) < hh:
                _ce(cv, ci, i, i + hh)
        hh //= 2
    return cv, ci


def _halve(vals, idxs, full=False):
    rows = vals[0].shape[0]
    h = rows // 2
    merge = _merge_full if full else _merge_keep
    mv, mi = merge([v[:h] for v in vals], [x[:h] for x in idxs],
                   [v[h:2 * h] for v in vals], [x[h:2 * h] for x in idxs])
    if rows % 2:
        if full:
            raise ValueError("odd rows unsupported for full merge")
        mv = [jnp.concatenate([v, vals[i][2 * h:]], axis=0)
              for i, v in enumerate(mv)]
        mi = [jnp.concatenate([x, idxs[i][2 * h:]], axis=0)
              for i, x in enumerate(mi)]
    return mv, mi


def _block_planes(nparts, n_real, m, refb, qtm2, r2, q2, base):
    crossm2 = jax.lax.dot_general(
        refb[...], qtm2[...], (((0,), (0,)), ((), ())),
        preferred_element_type=jnp.float32)
    nb = crossm2.shape[0]
    g = nb // nparts
    q2b = q2[...]
    giota = jax.lax.broadcasted_iota(jnp.int32, (g, m), 0)
    vals, idxs = [], []
    for r in range(nparts):
        lo = r * g
        d2r = (q2b + r2[lo:lo + g, :]) + crossm2[lo:lo + g, :]
        ir = giota + lo
        if n_real is not None:
            d2r = jnp.where(ir >= n_real - base, BIG, d2r)
        vals.append(d2r)
        idxs.append(ir)
    return vals, idxs


def _fast_body(nblocks, n_real, m, refb, qtm2, r2, q2, *outs):
    pi = pl.program_id(0)
    ov, oi = outs[:CAP], outs[CAP:]
    base = pi * NBF

    def run(mask_n):
        vals, idxs = _block_planes(CAP, mask_n, m, refb, qtm2, r2, q2,
                                   base)
        for (i, j) in _PAIRS4:
            _ce(vals, idxs, i, j)
        while vals[0].shape[0] > CLS:
            vals, idxs = _halve(vals, idxs)
        for r in range(CAP):
            ov[r][...] = vals[r]
            oi[r][...] = idxs[r] + base

    @pl.when(pi < nblocks - 1)
    def _():
        run(None)

    @pl.when(pi == nblocks - 1)
    def _():
        run(n_real)


def _collapse_body(m, iv0, iv1, iv2, iv3, ii0, ii1, ii2, ii3,
                   dout, iout, fout):
    av = [iv0[...], iv1[...], iv2[...], iv3[...]]
    ai = [ii0[...], ii1[...], ii2[...], ii3[...]]
    av, ai = _halve(av, ai, full=True)
    av, ai = _halve(av, ai, full=True)
    while av[0].shape[0] > 1:
        av, ai = _halve(av, ai)
    tau = av[K - 1]
    dout[...] = jnp.concatenate(
        [jnp.sqrt(jnp.maximum(v, 0.0)) for v in av], axis=0)
    iout[...] = jnp.concatenate(ai, axis=0)
    fire = (iv3[...] <= tau).astype(jnp.int32)
    fout[...] = jnp.max(fire, axis=0, keepdims=True)


def _exact_body(nblocks, g, n_real, m, refb, qtm2, r2, q2, dout, iout,
                sv, si):
    pi = pl.program_id(0)
    base = pi * NBX
    vals, idxs = _block_planes(K, n_real, m, refb, qtm2, r2, q2, base)
    idxs = [x + base for x in idxs]
    for (i, j) in _PAIRS16:
        _ce(vals, idxs, i, j)
    while vals[0].shape[0] > SROWS:
        vals, idxs = _halve(vals, idxs)

    @pl.when(pi == 0)
    def _():
        for i in range(K):
            sv[i * SROWS:(i + 1) * SROWS, :] = vals[i]
            si[i * SROWS:(i + 1) * SROWS, :] = idxs[i]

    @pl.when(pi > 0)
    def _():
        av = [sv[i * SROWS:(i + 1) * SROWS, :] for i in range(K)]
        ai = [si[i * SROWS:(i + 1) * SROWS, :] for i in range(K)]
        mv, mi = _merge_keep(av, ai, vals, idxs)
        for i in range(K):
            sv[i * SROWS:(i + 1) * SROWS, :] = mv[i]
            si[i * SROWS:(i + 1) * SROWS, :] = mi[i]

    @pl.when(pi == nblocks - 1)
    def _():
        fv = [sv[i * SROWS:(i + 1) * SROWS, :] for i in range(K)]
        fi = [si[i * SROWS:(i + 1) * SROWS, :] for i in range(K)]
        for (i, j) in _PAIRS16:
            _ce(fv, fi, i, j)
        while fv[0].shape[0] > 1:
            fv, fi = _halve(fv, fi)
        dout[...] = jnp.concatenate(
            [jnp.sqrt(jnp.maximum(v, 0.0)) for v in fv], axis=0)
        iout[...] = jnp.concatenate(fi, axis=0)


def kernel(ref, query):
    b, c, n = ref.shape
    m = query.shape[2]
    nbf = -(-n // NBF)
    nbx = -(-n // NBX)
    srows = nbf * CLS

    rp = jnp.transpose(ref, (0, 2, 1))
    qp = jnp.transpose(query, (0, 2, 1))
    r2col = jnp.sum(rp * rp, axis=-1)[0][:, None]
    q2row = jnp.sum(qp * qp, axis=-1)
    refsq = ref[0]
    qtm2 = -2.0 * query[0]

    vmem = pltpu.CompilerParams(vmem_limit_bytes=63 * 1024 * 1024)
    state = pl.pallas_call(
        functools.partial(_fast_body, nbf, n, m),
        grid=(nbf,),
        compiler_params=vmem,
        in_specs=[
            pl.BlockSpec((c, NBF), lambda i: (0, i)),
            pl.BlockSpec((c, m), lambda i: (0, 0)),
            pl.BlockSpec((NBF, 1), lambda i: (i, 0)),
            pl.BlockSpec((1, m), lambda i: (0, 0)),
        ],
        out_specs=[pl.BlockSpec((CLS, m), lambda i: (i, 0))] * (2 * CAP),
        out_shape=(
            [jax.ShapeDtypeStruct((srows, m), jnp.float32)] * CAP
            + [jax.ShapeDtypeStruct((srows, m), jnp.int32)] * CAP),
    )(refsq, qtm2, r2col, q2row)

    dist_f, idx_f, flags = pl.pallas_call(
        functools.partial(_collapse_body, m),
        compiler_params=vmem,
        out_shape=[
            jax.ShapeDtypeStruct((K, m), jnp.float32),
            jax.ShapeDtypeStruct((K, m), jnp.int32),
            jax.ShapeDtypeStruct((1, m), jnp.int32),
        ],
    )(*state)

    def _exact():
        return pl.pallas_call(
            functools.partial(_exact_body, nbx, NBX // K, n, m),
            grid=(nbx,),
            compiler_params=vmem,
            in_specs=[
                pl.BlockSpec((c, NBX), lambda i: (0, i)),
                pl.BlockSpec((c, m), lambda i: (0, 0)),
                pl.BlockSpec((NBX, 1), lambda i: (i, 0)),
                pl.BlockSpec((1, m), lambda i: (0, 0)),
            ],
            out_specs=[
                pl.BlockSpec((K, m), lambda i: (0, 0)),
                pl.BlockSpec((K, m), lambda i: (0, 0)),
            ],
            out_shape=[
                jax.ShapeDtypeStruct((K, m), jnp.float32),
                jax.ShapeDtypeStruct((K, m), jnp.int32),
            ],
            scratch_shapes=[
                pltpu.VMEM((K * SROWS, m), jnp.float32),
                pltpu.VMEM((K * SROWS, m), jnp.int32),
            ],
        )(refsq, qtm2, r2col, q2row)

    dist, idx = jax.lax.cond(jnp.any(flags > 0), _exact,
                             lambda: (dist_f, idx_f))
    return dist[None], idx[None]
